# Initial kernel scaffold; baseline (speedup 1.0000x reference)
import jax
import jax.numpy as jnp
from jax.experimental import pallas as pl


def kernel(x, edge_index_b0, edge_index_b1, edge_index_b2, edge_index_b3, edge_index_b4, edge_weight_b0, edge_weight_b1, edge_weight_b2, edge_weight_b3, edge_weight_b4, edge_index_struct, params):
    raise NotImplementedError("write your pallas kernel here")


# speedup vs baseline: 2.8524x; 2.8524x over previous
import functools

import jax
import jax.numpy as jnp
from jax.experimental import pallas as pl

NUM_BANDS = 5
NODES = 19


def _mlp_body(xc_ref, w1_ref, b1_ref, g3_ref, be3_ref, w2_ref, b2_ref,
              w3_ref, b3_ref, out_ref):
    xc = xc_ref[...]
    h = jnp.dot(xc, w1_ref[...], preferred_element_type=jnp.float32) + b1_ref[...]
    m = jnp.mean(h, axis=0, keepdims=True)
    v = jnp.mean((h - m) ** 2, axis=0, keepdims=True)
    h = (h - m) / jnp.sqrt(v + 1e-5) * g3_ref[...] + be3_ref[...]
    h = jnp.maximum(h, 0.0)
    h = jnp.maximum(jnp.dot(h, w2_ref[...], preferred_element_type=jnp.float32) + b2_ref[...], 0.0)
    out_ref[...] = jnp.dot(h, w3_ref[...], preferred_element_type=jnp.float32) + b3_ref[...]


def _mlp_head(xc, params):
    B = xc.shape[0]
    return pl.pallas_call(
        _mlp_body,
        out_shape=jax.ShapeDtypeStruct((B, 2), jnp.float32),
    )(xc, params['lin1_W'], params['lin1_b'].reshape(1, -1),
      params['bn3_g'].reshape(1, -1), params['bn3_b'].reshape(1, -1),
      params['lin2_W'], params['lin2_b'].reshape(1, -1),
      params['lin3_W'], params['lin3_b'].reshape(1, -1))


def kernel(x, edge_index_b0, edge_index_b1, edge_index_b2, edge_index_b3,
           edge_index_b4, edge_weight_b0, edge_weight_b1, edge_weight_b2,
           edge_weight_b3, edge_weight_b4, edge_index_struct, params):
    eis = (edge_index_b0, edge_index_b1, edge_index_b2, edge_index_b3, edge_index_b4)
    ews = (edge_weight_b0, edge_weight_b1, edge_weight_b2, edge_weight_b3, edge_weight_b4)
    n = x.shape[0]
    B = n // NODES

    Zcols = []
    for b in range(NUM_BANDS):
        src, dst = eis[b][0], eis[b][1]
        ew = ews[b]
        xb = x[:, b]
        deg = jnp.zeros((n,), x.dtype).at[dst].add(ew) + 1.0
        dinv = jnp.where(deg > 0, deg ** -0.5, 0.0)
        u = xb * dinv
        acc = jnp.zeros((n,), x.dtype).at[dst].add(u[src] * ew)
        s = dinv * acc + xb * dinv * dinv
        mu = jnp.mean(s)
        vs = jnp.mean((s - mu) ** 2)
        W1 = params['W1_%d' % b][0]
        alpha = W1 * params['g1_%d' % b] / jnp.sqrt(vs * W1 * W1 + 1e-5)
        g = jax.nn.relu((s - mu)[:, None] * alpha[None, :] + params['be1_%d' % b][None, :])
        Zcols.append(g @ params['W2_%d' % b])
    Z = jnp.concatenate(Zcols, axis=1)

    srcS, dstS = edge_index_struct[0], edge_index_struct[1]
    degS = jnp.zeros((n,), x.dtype).at[dstS].add(jnp.ones_like(dstS, x.dtype)) + 1.0
    dinvS = jnp.where(degS > 0, degS ** -0.5, 0.0)
    P = Z * dinvS[:, None]
    accS = jnp.zeros((n, 10), x.dtype).at[dstS].add(P[srcS])
    b2 = jnp.concatenate([params['b2_%d' % b] for b in range(NUM_BANDS)])
    out2 = dinvS[:, None] * (accS + P) + b2[None, :]
    m2 = jnp.mean(out2, axis=0)
    v2 = jnp.var(out2, axis=0)
    g2 = jnp.concatenate([params['g2_%d' % b] for b in range(NUM_BANDS)])
    be2 = jnp.concatenate([params['be2_%d' % b] for b in range(NUM_BANDS)])
    h2 = jax.nn.relu((out2 - m2) / jnp.sqrt(v2 + 1e-5) * g2 + be2)
    xc = h2.reshape(B, NODES, NUM_BANDS, 2).transpose(0, 2, 1, 3).reshape(B, -1)
    return _mlp_head(xc, params)


# speedup vs baseline: 102.0181x; 35.7662x over previous
import functools

import jax
import jax.numpy as jnp
from jax import lax
from jax.experimental import pallas as pl
from jax.experimental.pallas import tpu as pltpu
from jax.experimental.pallas import tpu_sc as plsc

N = 100016
E = 3200512
NUM_BANDS = 5
NODES = 19
NTILES = 32
EPW = E // NTILES
W = 5264
NWIN = EPW // W
CH = W // 16
BLKN = 2048
NBLK = -(-N // BLKN)

_f32 = jnp.float32


def _bf16_round(x):
    u = lax.bitcast_convert_type(x, jnp.uint32)
    r = (u + jnp.uint32(0x7FFF) + ((u >> 16) & jnp.uint32(1))) & jnp.uint32(0xFFFF0000)
    return lax.bitcast_convert_type(r, _f32)


def _mesh():
    return plsc.VectorSubcoreMesh(core_axis_name="c", subcore_axis_name="s",
                                  num_cores=2, num_subcores=16)


def _sc_params():
    return pltpu.CompilerParams(needs_layout_passes=False)


def _tile_id():
    return lax.axis_index("s") * 2 + lax.axis_index("c")


def _zero_acc(acc):
    def zb(i, _):
        acc[pl.ds(i * 16, 16)] = jnp.zeros((16,), _f32)
        return 0
    lax.fori_loop(0, N // 16, zb, 0)


def _scatter_pass(base, acc, idxb, valb, idx_hbm, val_hbm, val_off, out_hbm, t,
                  ones=False):
    _zero_acc(acc)

    def wb(w, _):
        off = base + w * W
        pltpu.sync_copy(idx_hbm.at[pl.ds(off, W)], idxb)
        if not ones:
            pltpu.sync_copy(val_hbm.at[pl.ds(val_off + off, W)], valb)

        def cb(i, _):
            sl = pl.ds(i * 16, 16)
            idx = idxb[sl]
            v = jnp.full((16,), 1.0, _f32) if ones else valb[sl]
            plsc.addupdate_scatter(acc, [idx], v)
            return 0
        lax.fori_loop(0, CH, cb, 0)
        return 0
    lax.fori_loop(0, NWIN, wb, 0)
    pltpu.sync_copy(acc, out_hbm.at[pl.ds(t * N, N)])


def _gather_pass(base, acc, idxb, valb, mbuf, tab_hbm, tab_off, src_hbm,
                 ew_hbm, m_hbm, m_off, unit=False):
    pltpu.sync_copy(tab_hbm.at[pl.ds(tab_off, N)], acc)

    def wb(w, _):
        off = base + w * W
        pltpu.sync_copy(src_hbm.at[pl.ds(off, W)], idxb)
        if not unit:
            pltpu.sync_copy(ew_hbm.at[pl.ds(off, W)], valb)

        def cb(i, _):
            sl = pl.ds(i * 16, 16)
            g = plsc.load_gather(acc, [idxb[sl]])
            mbuf[sl] = g if unit else g * valb[sl]
            return 0
        lax.fori_loop(0, CH, cb, 0)
        pltpu.sync_copy(mbuf, m_hbm.at[pl.ds(m_off + off, W)])
        return 0
    lax.fori_loop(0, NWIN, wb, 0)


def _sc_deg_body(*refs):
    ins = refs[:11]
    outs = refs[11:17]
    acc, idxb, valb = refs[17:]
    dsts, ews, dstS = ins[0:5], ins[5:10], ins[10]
    t = _tile_id()
    base = t * EPW
    for b in range(NUM_BANDS):
        _scatter_pass(base, acc, idxb, valb, dsts[b], ews[b], 0, outs[b], t)
    _scatter_pass(base, acc, idxb, valb, dstS, None, 0, outs[5], t, ones=True)


def _sc_band_body(*refs):
    ut = refs[0]
    srcs, dsts, ews = refs[1:6], refs[6:11], refs[11:16]
    m = refs[16]
    outs = refs[17:22]
    acc, idxb, valb, mbuf = refs[22:]
    t = _tile_id()
    base = t * EPW
    for b in range(NUM_BANDS):
        _gather_pass(base, acc, idxb, valb, mbuf, ut, b * N, srcs[b], ews[b],
                     m, b * E)
        _scatter_pass(base, acc, idxb, valb, dsts[b], m, b * E, outs[b], t)


def _sc_struct_body(*refs):
    pt, srcS, dstS, m = refs[0], refs[1], refs[2], refs[3]
    outs = refs[4:14]
    acc, idxb, valb, mbuf = refs[14:]
    t = _tile_id()
    base = t * EPW
    for c in range(10):
        _gather_pass(base, acc, idxb, valb, mbuf, pt, c * N, srcS, None,
                     m, c * E, unit=True)
        _scatter_pass(base, acc, idxb, valb, dstS, m, c * E, outs[c], t)


def _sc_deg(dsts, ews, dstS):
    f = pl.kernel(
        _sc_deg_body,
        out_type=[jax.ShapeDtypeStruct((NTILES * N,), _f32)] * 6,
        mesh=_mesh(),
        compiler_params=_sc_params(),
        scratch_types=[pltpu.VMEM((N,), _f32), pltpu.VMEM((W,), jnp.int32),
                       pltpu.VMEM((W,), _f32)],
    )
    return f(*dsts, *ews, dstS)


def _sc_band(ut, srcs, dsts, ews):
    f = pl.kernel(
        _sc_band_body,
        out_type=[jax.ShapeDtypeStruct((NUM_BANDS * E,), _f32)]
        + [jax.ShapeDtypeStruct((NTILES * N,), _f32)] * 5,
        mesh=_mesh(),
        compiler_params=_sc_params(),
        scratch_types=[pltpu.VMEM((N,), _f32), pltpu.VMEM((W,), jnp.int32),
                       pltpu.VMEM((W,), _f32), pltpu.VMEM((W,), _f32)],
    )
    return f(ut, *srcs, *dsts, *ews)


def _sc_struct(pt, srcS, dstS):
    f = pl.kernel(
        _sc_struct_body,
        out_type=[jax.ShapeDtypeStruct((10 * E,), _f32)]
        + [jax.ShapeDtypeStruct((NTILES * N,), _f32)] * 10,
        mesh=_mesh(),
        compiler_params=_sc_params(),
        scratch_types=[pltpu.VMEM((N,), _f32), pltpu.VMEM((W,), jnp.int32),
                       pltpu.VMEM((W,), _f32), pltpu.VMEM((W,), _f32)],
    )
    return f(pt, srcS, dstS)


def _tc_a_body(*refs):
    degs = refs[0:6]
    xt = refs[6]
    ut_o, dt_o = refs[7], refs[8]
    dinvs = []
    for p in range(6):
        deg = jnp.sum(degs[p][...], axis=0, keepdims=True) + 1.0
        dinvs.append(jnp.where(deg > 0, lax.rsqrt(deg), 0.0))
    dt_o[...] = jnp.concatenate(dinvs, axis=0)
    ut_o[...] = xt[...] * jnp.concatenate(dinvs[:5], axis=0)


def _tc_a(degparts, x_t):
    grid = (NBLK,)
    R = degparts[0].shape[0]
    return pl.pallas_call(
        _tc_a_body,
        grid=grid,
        in_specs=[pl.BlockSpec((R, BLKN), lambda i: (0, i))] * 6
        + [pl.BlockSpec((5, BLKN), lambda i: (0, i))],
        out_specs=[pl.BlockSpec((5, BLKN), lambda i: (0, i)),
                   pl.BlockSpec((6, BLKN), lambda i: (0, i))],
        out_shape=[jax.ShapeDtypeStruct((5, N), _f32),
                   jax.ShapeDtypeStruct((6, N), _f32)],
    )(*degparts, x_t)


def _tc_b_body(*refs):
    accs = refs[0:5]
    dt, xt = refs[5], refs[6]
    st_o, stats_o = refs[7], refs[8]
    i = pl.program_id(0)
    col = jax.lax.broadcasted_iota(jnp.int32, (1, BLKN), 1) + i * BLKN
    mask = col < N
    rows = []
    for b in range(NUM_BANDS):
        acc = jnp.sum(accs[b][...], axis=0, keepdims=True)
        dinv = dt[b:b + 1, :]
        s = dinv * acc + xt[b:b + 1, :] * dinv * dinv
        rows.append(s)
    st_o[...] = jnp.concatenate(rows, axis=0)

    @pl.when(i == 0)
    def _():
        stats_o[...] = jnp.zeros((16, 128), _f32)
        c = jnp.concatenate(
            [jnp.sum(r, axis=1, keepdims=True) / BLKN for r in rows], axis=0)
        stats_o[10:15, 0:1] = c
    cvec = stats_o[10:15, 0:1]
    s1, s2 = [], []
    for b in range(NUM_BANDS):
        sm = jnp.where(mask, rows[b] - cvec[b:b + 1, :], 0.0)
        s1.append(jnp.sum(sm, axis=1, keepdims=True))
        s2.append(jnp.sum(sm * sm, axis=1, keepdims=True))
    upd = jnp.concatenate(
        [jnp.concatenate(s1 + s2, axis=0), jnp.zeros((10, 127), _f32)], axis=1)
    stats_o[0:10, :] += upd


def _tc_b(accparts, d_t, x_t):
    R = accparts[0].shape[0]
    return pl.pallas_call(
        _tc_b_body,
        grid=(NBLK,),
        in_specs=[pl.BlockSpec((R, BLKN), lambda i: (0, i))] * 5
        + [pl.BlockSpec((6, BLKN), lambda i: (0, i)),
           pl.BlockSpec((5, BLKN), lambda i: (0, i))],
        out_specs=[pl.BlockSpec((5, BLKN), lambda i: (0, i)),
                   pl.BlockSpec((16, 128), lambda i: (0, 0))],
        out_shape=[jax.ShapeDtypeStruct((5, N), _f32),
                   jax.ShapeDtypeStruct((16, 128), _f32)],
    )(*accparts, d_t, x_t)


def _tc_b2_body(st, dt, stats, w1t, g1t, be1t, w2t, pt_o):
    rows = []
    dinvS = dt[5:6, :]
    for b in range(NUM_BANDS):
        c = stats[10 + b, 0]
        d = jnp.sum(stats[b, :]) / N
        mu = c + d
        var = jnp.sum(stats[5 + b, :]) / N - d * d
        w1 = w1t[:, b:b + 1]
        alpha = w1 * g1t[:, b:b + 1] * lax.rsqrt(var * w1 * w1 + 1e-5)
        g = jnp.maximum((st[b:b + 1, :] - mu) * alpha + be1t[:, b:b + 1], 0.0)
        for j in range(2):
            w2 = w2t[:, 2 * b + j:2 * b + j + 1]
            rows.append(jnp.sum(g * w2, axis=0, keepdims=True) * dinvS)
    rows.append(jnp.zeros((6, BLKN), _f32))
    pt_o[...] = jnp.concatenate(rows, axis=0)


def _tc_b2(s_t, d_t, stats, w1t, g1t, be1t, w2t):
    return pl.pallas_call(
        _tc_b2_body,
        grid=(NBLK,),
        in_specs=[pl.BlockSpec((5, BLKN), lambda i: (0, i)),
                  pl.BlockSpec((6, BLKN), lambda i: (0, i)),
                  pl.BlockSpec((16, 128), lambda i: (0, 0)),
                  pl.BlockSpec((16, 5), lambda i: (0, 0)),
                  pl.BlockSpec((16, 5), lambda i: (0, 0)),
                  pl.BlockSpec((16, 5), lambda i: (0, 0)),
                  pl.BlockSpec((16, 10), lambda i: (0, 0))],
        out_specs=pl.BlockSpec((16, BLKN), lambda i: (0, i)),
        out_shape=jax.ShapeDtypeStruct((16, N), _f32),
    )(s_t, d_t, stats, w1t, g1t, be1t, w2t)


def _tc_c_body(*refs):
    sparts = refs[0:10]
    pt, dt, b2t = refs[10], refs[11], refs[12]
    o_o, stats_o = refs[13], refs[14]
    i = pl.program_id(0)
    col = jax.lax.broadcasted_iota(jnp.int32, (1, BLKN), 1) + i * BLKN
    mask = col < N
    dinvS = dt[5:6, :]
    rows = []
    for r in range(10):
        accS = jnp.sum(sparts[r][...], axis=0, keepdims=True)
        o = dinvS * (accS + pt[r:r + 1, :]) + b2t[r:r + 1, :]
        rows.append(o)
    o_o[...] = jnp.concatenate(rows, axis=0)

    @pl.when(i == 0)
    def _():
        stats_o[...] = jnp.zeros((32, 128), _f32)
        c = jnp.concatenate(
            [jnp.sum(r, axis=1, keepdims=True) / BLKN for r in rows], axis=0)
        stats_o[20:30, 0:1] = c
    cvec = stats_o[20:30, 0:1]
    s1, s2 = [], []
    for r in range(10):
        om = jnp.where(mask, rows[r] - cvec[r:r + 1, :], 0.0)
        s1.append(jnp.sum(om, axis=1, keepdims=True))
        s2.append(jnp.sum(om * om, axis=1, keepdims=True))
    upd = jnp.concatenate(
        [jnp.concatenate(s1 + s2, axis=0), jnp.zeros((20, 127), _f32)], axis=1)
    stats_o[0:20, :] += upd


def _tc_c(sparts, p_t, d_t, b2t):
    R = sparts[0].shape[0]
    return pl.pallas_call(
        _tc_c_body,
        grid=(NBLK,),
        in_specs=[pl.BlockSpec((R, BLKN), lambda i: (0, i))] * 10
        + [pl.BlockSpec((16, BLKN), lambda i: (0, i)),
           pl.BlockSpec((6, BLKN), lambda i: (0, i)),
           pl.BlockSpec((10, 1), lambda i: (0, 0))],
        out_specs=[pl.BlockSpec((10, BLKN), lambda i: (0, i)),
                   pl.BlockSpec((32, 128), lambda i: (0, 0))],
        out_shape=[jax.ShapeDtypeStruct((10, N), _f32),
                   jax.ShapeDtypeStruct((32, 128), _f32)],
    )(*sparts, p_t, d_t, b2t)


def _tc_c2_body(ot, stats, g2t, be2t, h_o):
    c = stats[20:30, 0:1]
    d = jnp.sum(stats[0:10, :], axis=1, keepdims=True) / N
    mu = c + d
    var = jnp.sum(stats[10:20, :], axis=1, keepdims=True) / N - d * d
    h_o[...] = _bf16_round(jnp.maximum(
        (ot[...] - mu) * lax.rsqrt(var + 1e-5) * g2t[...] + be2t[...], 0.0))


def _tc_c2(o_t, stats2, g2t, be2t):
    return pl.pallas_call(
        _tc_c2_body,
        grid=(NBLK,),
        in_specs=[pl.BlockSpec((10, BLKN), lambda i: (0, i)),
                  pl.BlockSpec((32, 128), lambda i: (0, 0)),
                  pl.BlockSpec((10, 1), lambda i: (0, 0)),
                  pl.BlockSpec((10, 1), lambda i: (0, 0))],
        out_specs=pl.BlockSpec((10, BLKN), lambda i: (0, i)),
        out_shape=jax.ShapeDtypeStruct((10, N), _f32),
    )(o_t, stats2, g2t, be2t)


def _mlp_body(xc_ref, w1_ref, b1_ref, g3_ref, be3_ref, w2_ref, b2_ref,
              w3_ref, b3_ref, out_ref):
    xc = xc_ref[...]
    h = jnp.dot(xc, w1_ref[...], preferred_element_type=_f32) + b1_ref[...]
    m = jnp.mean(h, axis=0, keepdims=True)
    v = jnp.mean((h - m) ** 2, axis=0, keepdims=True)
    h = (h - m) / jnp.sqrt(v + 1e-5) * g3_ref[...] + be3_ref[...]
    h = jnp.maximum(h, 0.0)
    h = jnp.maximum(jnp.dot(h, w2_ref[...], preferred_element_type=_f32)
                    + b2_ref[...], 0.0)
    h = _bf16_round(h)
    out_ref[...] = jnp.dot(h, w3_ref[...], preferred_element_type=_f32) + b3_ref[...]


def _mlp_head(xc, params):
    B = xc.shape[0]
    return pl.pallas_call(
        _mlp_body,
        out_shape=jax.ShapeDtypeStruct((B, 2), _f32),
    )(xc, params['lin1_W'], params['lin1_b'].reshape(1, -1),
      params['bn3_g'].reshape(1, -1), params['bn3_b'].reshape(1, -1),
      params['lin2_W'], params['lin2_b'].reshape(1, -1),
      params['lin3_W'], params['lin3_b'].reshape(1, -1))


_SC_DEG = True
_SC_BAND = True
_SC_STRUCT = True
_TC_JNP = True


def kernel(x, edge_index_b0, edge_index_b1, edge_index_b2, edge_index_b3,
           edge_index_b4, edge_weight_b0, edge_weight_b1, edge_weight_b2,
           edge_weight_b3, edge_weight_b4, edge_index_struct, params):
    eis = (edge_index_b0, edge_index_b1, edge_index_b2, edge_index_b3,
           edge_index_b4)
    ews = [edge_weight_b0, edge_weight_b1, edge_weight_b2, edge_weight_b3,
           edge_weight_b4]
    srcs = [e[0] for e in eis]
    dsts = [e[1] for e in eis]
    srcS, dstS = edge_index_struct[0], edge_index_struct[1]
    B = N // NODES
    x_t = x.T

    if _SC_DEG:
        degparts = _sc_deg(dsts, ews, dstS)
        degparts = [p.reshape(NTILES, N) for p in degparts]
    else:
        degparts = [jnp.zeros((N,), _f32).at[dsts[b]].add(ews[b]).reshape(1, N)
                    for b in range(5)]
        degparts.append(jnp.zeros((N,), _f32).at[dstS].add(
            jnp.ones_like(dstS, _f32)).reshape(1, N))

    if _TC_JNP:
        degs = [jnp.sum(pp, axis=0) + 1.0 for pp in degparts]
        dinvs = [jnp.where(dg > 0, dg ** -0.5, 0.0) for dg in degs]
        d_t = jnp.stack(dinvs, axis=0)
        u_t = x_t * d_t[:5]
    else:
        u_t, d_t = _tc_a(degparts, x_t)

    if _SC_BAND:
        band_out = _sc_band(u_t.reshape(-1), srcs, dsts, ews)
        accparts = [p.reshape(NTILES, N) for p in band_out[1:]]
    else:
        accparts = [jnp.zeros((N,), _f32).at[dsts[b]].add(
            u_t[b][srcs[b]] * ews[b]).reshape(1, N) for b in range(5)]

    if _TC_JNP:
        zcols = []
        for b in range(5):
            acc = jnp.sum(accparts[b], axis=0)
            s = d_t[b] * acc + x_t[b] * d_t[b] * d_t[b]
            h = s[:, None] * params['W1_%d' % b][0][None, :] + params['b1_%d' % b][None, :]
            mh = jnp.mean(h, axis=0)
            vh = jnp.var(h, axis=0)
            g = jax.nn.relu((h - mh) / jnp.sqrt(vh + 1e-5)
                            * params['g1_%d' % b] + params['be1_%d' % b])
            zcols.append(g @ params['W2_%d' % b])
        p_t = (jnp.concatenate(zcols, axis=1) * d_t[5][:, None]).T
    else:
        s_t, stats1 = _tc_b(accparts, d_t, x_t)
        w1t = jnp.stack([params['W1_%d' % b][0] for b in range(5)], axis=1)
        g1t = jnp.stack([params['g1_%d' % b] for b in range(5)], axis=1)
        be1t = jnp.stack([params['be1_%d' % b] for b in range(5)], axis=1)
        w2t = jnp.concatenate([params['W2_%d' % b] for b in range(5)], axis=1)
        p_t = _tc_b2(s_t, d_t, stats1, w1t, g1t, be1t, w2t)

    if _SC_STRUCT:
        struct_out = _sc_struct(p_t.reshape(-1), srcS, dstS)
        sparts = [p.reshape(NTILES, N) for p in struct_out[1:]]
    else:
        sparts = [jnp.zeros((N,), _f32).at[dstS].add(
            p_t[c][srcS]).reshape(1, N) for c in range(10)]

    b2t = jnp.concatenate([params['b2_%d' % b] for b in range(5)]).reshape(10, 1)
    g2t = jnp.concatenate([params['g2_%d' % b] for b in range(5)]).reshape(10, 1)
    be2t = jnp.concatenate([params['be2_%d' % b] for b in range(5)]).reshape(10, 1)
    if _TC_JNP:
        accS = jnp.stack([jnp.sum(pp, axis=0) for pp in sparts], axis=1)
        out2 = d_t[5][:, None] * (accS + p_t[:10].T) + b2t.reshape(1, 10)
        m2 = jnp.mean(out2, axis=0)
        v2 = jnp.var(out2, axis=0)
        h2 = jax.nn.relu((out2 - m2) / jnp.sqrt(v2 + 1e-5)
                         * g2t.reshape(1, 10) + be2t.reshape(1, 10))
    else:
        o_t, stats2 = _tc_c(sparts, p_t, d_t, b2t)
        h2_t = _tc_c2(o_t, stats2, g2t, be2t)

        h2 = h2_t.T
    xc = h2.reshape(B, NODES, NUM_BANDS, 2).transpose(0, 2, 1, 3).reshape(B, -1)
    return _mlp_head(xc, params)


# speedup vs baseline: 102.0816x; 1.0006x over previous
import jax
import jax.numpy as jnp
from jax import lax
from jax.experimental import pallas as pl
from jax.experimental.pallas import tpu as pltpu
from jax.experimental.pallas import tpu_sc as plsc

N = 100016
E = 3200512
NUM_BANDS = 5
NODES = 19
NTILES = 32
EPW = E // NTILES
W = 5264
NWIN = EPW // W
CH = W // 16

_f32 = jnp.float32


def _bf16_round(x):
    u = lax.bitcast_convert_type(x, jnp.uint32)
    r = (u + jnp.uint32(0x7FFF) + ((u >> 16) & jnp.uint32(1))) & jnp.uint32(0xFFFF0000)
    return lax.bitcast_convert_type(r, _f32)


def _mesh():
    return plsc.VectorSubcoreMesh(core_axis_name="c", subcore_axis_name="s",
                                  num_cores=2, num_subcores=16)


def _sc_params():
    return pltpu.CompilerParams(needs_layout_passes=False)


def _tile_id():
    return lax.axis_index("s") * 2 + lax.axis_index("c")


def _zero_acc(acc):
    def zb(i, _):
        acc[pl.ds(i * 16, 16)] = jnp.zeros((16,), _f32)
        return 0
    lax.fori_loop(0, N // 16, zb, 0)


def _scatter_pass(base, acc, idxb, valb, idx_hbm, val_hbm, val_off, out_hbm, t,
                  ones=False):
    _zero_acc(acc)

    def wb(w, _):
        off = base + w * W
        pltpu.sync_copy(idx_hbm.at[pl.ds(off, W)], idxb)
        if not ones:
            pltpu.sync_copy(val_hbm.at[pl.ds(val_off + off, W)], valb)

        def cb(i, _):
            sl = pl.ds(i * 16, 16)
            idx = idxb[sl]
            v = jnp.full((16,), 1.0, _f32) if ones else valb[sl]
            plsc.addupdate_scatter(acc, [idx], v)
            return 0
        lax.fori_loop(0, CH, cb, 0)
        return 0
    lax.fori_loop(0, NWIN, wb, 0)
    pltpu.sync_copy(acc, out_hbm.at[pl.ds(t * N, N)])


def _gather_pass(base, acc, idxb, valb, mbuf, tab_hbm, tab_off, src_hbm,
                 ew_hbm, m_hbm, m_off, unit=False):
    pltpu.sync_copy(tab_hbm.at[pl.ds(tab_off, N)], acc)

    def wb(w, _):
        off = base + w * W
        pltpu.sync_copy(src_hbm.at[pl.ds(off, W)], idxb)
        if not unit:
            pltpu.sync_copy(ew_hbm.at[pl.ds(off, W)], valb)

        def cb(i, _):
            sl = pl.ds(i * 16, 16)
            g = plsc.load_gather(acc, [idxb[sl]])
            mbuf[sl] = g if unit else g * valb[sl]
            return 0
        lax.fori_loop(0, CH, cb, 0)
        pltpu.sync_copy(mbuf, m_hbm.at[pl.ds(m_off + off, W)])
        return 0
    lax.fori_loop(0, NWIN, wb, 0)


def _sc_deg_body(*refs):
    ins = refs[:11]
    outs = refs[11:17]
    acc, idxb, valb = refs[17:]
    dsts, ews, dstS = ins[0:5], ins[5:10], ins[10]
    t = _tile_id()
    base = t * EPW
    for b in range(NUM_BANDS):
        _scatter_pass(base, acc, idxb, valb, dsts[b], ews[b], 0, outs[b], t)
    _scatter_pass(base, acc, idxb, valb, dstS, None, 0, outs[5], t, ones=True)


def _sc_band_body(*refs):
    ut = refs[0]
    srcs, dsts, ews = refs[1:6], refs[6:11], refs[11:16]
    m = refs[16]
    outs = refs[17:22]
    acc, idxb, valb, mbuf = refs[22:]
    t = _tile_id()
    base = t * EPW
    for b in range(NUM_BANDS):
        _gather_pass(base, acc, idxb, valb, mbuf, ut, b * N, srcs[b], ews[b],
                     m, b * E)
        _scatter_pass(base, acc, idxb, valb, dsts[b], m, b * E, outs[b], t)


def _sc_struct_body(*refs):
    pt, srcS, dstS, m = refs[0], refs[1], refs[2], refs[3]
    outs = refs[4:14]
    acc, idxb, valb, mbuf = refs[14:]
    t = _tile_id()
    base = t * EPW
    for c in range(10):
        _gather_pass(base, acc, idxb, valb, mbuf, pt, c * N, srcS, None,
                     m, c * E, unit=True)
        _scatter_pass(base, acc, idxb, valb, dstS, m, c * E, outs[c], t)


def _sc_deg(dsts, ews, dstS):
    f = pl.kernel(
        _sc_deg_body,
        out_type=[jax.ShapeDtypeStruct((NTILES * N,), _f32)] * 6,
        mesh=_mesh(),
        compiler_params=_sc_params(),
        scratch_types=[pltpu.VMEM((N,), _f32), pltpu.VMEM((W,), jnp.int32),
                       pltpu.VMEM((W,), _f32)],
    )
    return f(*dsts, *ews, dstS)


def _sc_band(ut, srcs, dsts, ews):
    f = pl.kernel(
        _sc_band_body,
        out_type=[jax.ShapeDtypeStruct((NUM_BANDS * E,), _f32)]
        + [jax.ShapeDtypeStruct((NTILES * N,), _f32)] * 5,
        mesh=_mesh(),
        compiler_params=_sc_params(),
        scratch_types=[pltpu.VMEM((N,), _f32), pltpu.VMEM((W,), jnp.int32),
                       pltpu.VMEM((W,), _f32), pltpu.VMEM((W,), _f32)],
    )
    return f(ut, *srcs, *dsts, *ews)


def _sc_struct(pt, srcS, dstS):
    f = pl.kernel(
        _sc_struct_body,
        out_type=[jax.ShapeDtypeStruct((10 * E,), _f32)]
        + [jax.ShapeDtypeStruct((NTILES * N,), _f32)] * 10,
        mesh=_mesh(),
        compiler_params=_sc_params(),
        scratch_types=[pltpu.VMEM((N,), _f32), pltpu.VMEM((W,), jnp.int32),
                       pltpu.VMEM((W,), _f32), pltpu.VMEM((W,), _f32)],
    )
    return f(pt, srcS, dstS)


def _mlp_body(xc_ref, w1_ref, b1_ref, g3_ref, be3_ref, w2_ref, b2_ref,
              w3_ref, b3_ref, out_ref):
    xc = xc_ref[...]
    h = jnp.dot(xc, w1_ref[...], preferred_element_type=_f32) + b1_ref[...]
    m = jnp.mean(h, axis=0, keepdims=True)
    v = jnp.mean((h - m) ** 2, axis=0, keepdims=True)
    h = (h - m) / jnp.sqrt(v + 1e-5) * g3_ref[...] + be3_ref[...]
    h = jnp.maximum(h, 0.0)
    h = jnp.maximum(jnp.dot(h, w2_ref[...], preferred_element_type=_f32)
                    + b2_ref[...], 0.0)
    h = _bf16_round(h)
    out_ref[...] = jnp.dot(h, w3_ref[...], preferred_element_type=_f32) + b3_ref[...]


def _mlp_head(xc, params):
    B = xc.shape[0]
    return pl.pallas_call(
        _mlp_body,
        out_shape=jax.ShapeDtypeStruct((B, 2), _f32),
    )(xc, params['lin1_W'], params['lin1_b'].reshape(1, -1),
      params['bn3_g'].reshape(1, -1), params['bn3_b'].reshape(1, -1),
      params['lin2_W'], params['lin2_b'].reshape(1, -1),
      params['lin3_W'], params['lin3_b'].reshape(1, -1))


def kernel(x, edge_index_b0, edge_index_b1, edge_index_b2, edge_index_b3,
           edge_index_b4, edge_weight_b0, edge_weight_b1, edge_weight_b2,
           edge_weight_b3, edge_weight_b4, edge_index_struct, params):
    eis = (edge_index_b0, edge_index_b1, edge_index_b2, edge_index_b3,
           edge_index_b4)
    ews = [edge_weight_b0, edge_weight_b1, edge_weight_b2, edge_weight_b3,
           edge_weight_b4]
    srcs = [e[0] for e in eis]
    dsts = [e[1] for e in eis]
    srcS, dstS = edge_index_struct[0], edge_index_struct[1]
    B = N // NODES
    x_t = x.T

    degparts = _sc_deg(dsts, ews, dstS)
    degparts = [p.reshape(NTILES, N) for p in degparts]

    degs = [jnp.sum(pp, axis=0) + 1.0 for pp in degparts]
    dinvs = [jnp.where(dg > 0, dg ** -0.5, 0.0) for dg in degs]
    d_t = jnp.stack(dinvs, axis=0)
    u_t = x_t * d_t[:5]

    band_out = _sc_band(u_t.reshape(-1), srcs, dsts, ews)
    accparts = [p.reshape(NTILES, N) for p in band_out[1:]]

    zcols = []
    for b in range(NUM_BANDS):
        acc = jnp.sum(accparts[b], axis=0)
        s = d_t[b] * acc + x_t[b] * d_t[b] * d_t[b]
        h = s[:, None] * params['W1_%d' % b][0][None, :] + params['b1_%d' % b][None, :]
        mh = jnp.mean(h, axis=0)
        vh = jnp.var(h, axis=0)
        g = jax.nn.relu((h - mh) / jnp.sqrt(vh + 1e-5)
                        * params['g1_%d' % b] + params['be1_%d' % b])
        zcols.append(g @ params['W2_%d' % b])
    p_t = (jnp.concatenate(zcols, axis=1) * d_t[5][:, None]).T

    struct_out = _sc_struct(p_t.reshape(-1), srcS, dstS)
    sparts = [p.reshape(NTILES, N) for p in struct_out[1:]]

    b2t = jnp.concatenate([params['b2_%d' % b] for b in range(5)])
    g2t = jnp.concatenate([params['g2_%d' % b] for b in range(5)])
    be2t = jnp.concatenate([params['be2_%d' % b] for b in range(5)])
    accS = jnp.stack([jnp.sum(pp, axis=0) for pp in sparts], axis=1)
    out2 = d_t[5][:, None] * (accS + p_t.T) + b2t.reshape(1, 10)
    m2 = jnp.mean(out2, axis=0)
    v2 = jnp.var(out2, axis=0)
    h2 = jax.nn.relu((out2 - m2) / jnp.sqrt(v2 + 1e-5)
                     * g2t.reshape(1, 10) + be2t.reshape(1, 10))

    xc = h2.reshape(B, NODES, NUM_BANDS, 2).transpose(0, 2, 1, 3).reshape(B, -1)
    return _mlp_head(xc, params)
